# baseline (speedup 1.0000x reference)
import functools

import jax
import jax.numpy as jnp
from jax import lax
from jax.experimental import pallas as pl
from jax.experimental.pallas import tpu as pltpu
from jax.experimental.pallas import tpu_sc as plsc

NUM_EXPERTS = 16
CAPACITY = 64
LOAD_PENALTY = 2.0
DIVERSITY_TEMP = 2.0
ALPHA = 0.6
BN_EPS = 1e-5
B = 256
NCLS = 10
NCHUNK = 16
NROUNDS = 4


def _conv(x, w, b):
    y = lax.conv_general_dilated(x, w, (1, 1), 'SAME',
                                 dimension_numbers=('NCHW', 'OIHW', 'NCHW'))
    return y + b[None, :, None, None]


def _bn(x, g, b, m, v):
    return (x - m[None, :, None, None]) / jnp.sqrt(v[None, :, None, None] + BN_EPS) \
        * g[None, :, None, None] + b[None, :, None, None]


def _maxpool(x):
    return lax.reduce_window(x, -jnp.inf, lax.max, (1, 1, 2, 2), (1, 1, 2, 2), 'VALID')


def _trunk(x, p):
    h = jax.nn.relu(_bn(_conv(x, p['c1w'], p['c1b']), p['g1'], p['be1'], p['m1'], p['v1']))
    h = jax.nn.relu(_bn(_conv(h, p['c2w'], p['c2b']), p['g2'], p['be2'], p['m2'], p['v2']))
    h = _maxpool(h)
    h = jax.nn.relu(_bn(_conv(h, p['c3w'], p['c3b']), p['g3'], p['be3'], p['m3'], p['v3']))
    h = jax.nn.relu(_bn(_conv(h, p['c4w'], p['c4b']), p['g4'], p['be4'], p['m4'], p['v4']))
    h = _maxpool(h)
    return h.mean(axis=(2, 3))


def _heads_body(ftT_ref, cwT_ref, cb_ref, gw1_ref, gb1_ref, g2m_ref, gb2_ref,
                ema_ref, lgT_ref, rsT_ref):
    ftT = ftT_ref[...]
    lgT = jnp.dot(cwT_ref[...], ftT,
                  preferred_element_type=jnp.float32) + cb_ref[...]
    lgT_ref[...] = lgT
    lg3 = lgT.reshape(NUM_EXPERTS, NCLS, B)
    m = jnp.max(lg3, axis=1, keepdims=True)
    p = jnp.exp(lg3 - m)
    p = p / jnp.sum(p, axis=1, keepdims=True)
    confT = jnp.sum(p * jnp.log(jnp.clip(p, 1e-12)), axis=1)
    hT = jnp.maximum(jnp.dot(gw1_ref[...], ftT,
                             preferred_element_type=jnp.float32) + gb1_ref[...], 0.0)
    esT = (jnp.dot(g2m_ref[...], hT,
                   preferred_element_type=jnp.float32) + gb2_ref[...]) / DIVERSITY_TEMP
    rsT_ref[...] = ALPHA * esT + (1.0 - ALPHA) * confT - LOAD_PENALTY * ema_ref[...]


def _heads(ftT, p, interpret=False):
    D = ftT.shape[0]
    GH = p['gW1'].shape[1]
    cwT = p['cls_w'].reshape(NUM_EXPERTS * NCLS, D)
    cb = p['cls_b'].reshape(NUM_EXPERTS * NCLS, 1)
    gw1 = p['gW1'].reshape(NUM_EXPERTS * GH, D)
    gb1 = p['gb1'].reshape(NUM_EXPERTS * GH, 1)
    g2m = (jnp.eye(NUM_EXPERTS, dtype=jnp.float32)[:, :, None]
           * p['gW2'][:, 0, :][None]).reshape(NUM_EXPERTS, NUM_EXPERTS * GH)
    gb2 = p['gb2']
    ema = p['ema'].reshape(NUM_EXPERTS, 1)
    return pl.pallas_call(
        _heads_body,
        out_shape=(
            jax.ShapeDtypeStruct((NUM_EXPERTS * NCLS, B), jnp.float32),
            jax.ShapeDtypeStruct((NUM_EXPERTS, B), jnp.float32),
        ),
        interpret=interpret,
    )(ftT, cwT, cb, gw1, gb1, g2m, gb2, ema)


_NEG_INF = float('-inf')


@functools.cache
def _make_sc_route():
    mesh = plsc.VectorSubcoreMesh(core_axis_name="c", subcore_axis_name="s")
    return functools.partial(
        pl.kernel,
        mesh=mesh,
        out_type=jax.ShapeDtypeStruct((NROUNDS * B,), jnp.float32),
        scratch_types=[
            pltpu.VMEM((B,), jnp.float32),
            pltpu.VMEM((B,), jnp.float32),
            pltpu.VMEM((B,), jnp.float32),
            pltpu.VMEM((NROUNDS * NCHUNK,), jnp.float32),
            pltpu.VMEM_SHARED((B,), jnp.float32),
        ],
    )(_sc_route_body)


def _sc_route(rsT):
    return _make_sc_route()(rsT)


def _sc_route_body(rst_hbm, df_hbm, scores_v, smask_v, avail_v, df_v, shared_avail):
    s = lax.axis_index("s")
    c = lax.axis_index("c")
    base = s * NCHUNK
    lane = lax.iota(jnp.int32, NCHUNK)
    tok_idx = base + lane

    for q in range(B // NCHUNK):
        sl = pl.ds(q * NCHUNK, NCHUNK)
        avail_v[sl] = jnp.ones((NCHUNK,), jnp.float32)

    for j in range(NROUNDS):
        pltpu.sync_copy(rst_hbm.at[j], scores_v)
        for q in range(B // NCHUNK):
            sl = pl.ds(q * NCHUNK, NCHUNK)
            smask_v[sl] = jnp.where(avail_v[sl] > 0.0, scores_v[sl], _NEG_INF)
        vi = scores_v[pl.ds(base, NCHUNK)]
        av_i = avail_v[pl.ds(base, NCHUNK)]

        def body(q, cnt):
            vk = smask_v[pl.ds(q * NCHUNK, NCHUNK)]
            kbase = q * NCHUNK
            for l in range(NCHUNK):
                sk = vk[l]
                k = kbase + l
                beats = (sk > vi) | ((sk == vi) & (k < tok_idx))
                cnt = cnt + jnp.where(beats, 1, 0)
            return cnt

        cnt = lax.fori_loop(0, B // NCHUNK, body, jnp.zeros((NCHUNK,), jnp.int32))
        sel = jnp.where((av_i > 0.0) & (cnt < CAPACITY), 1.0, 0.0)
        df_v[pl.ds(j * NCHUNK, NCHUNK)] = sel
        avail_v[pl.ds(base, NCHUNK)] = av_i - sel
        pltpu.sync_copy(avail_v.at[pl.ds(base, NCHUNK)],
                        shared_avail.at[pl.ds(base, NCHUNK)])
        plsc.subcore_barrier()
        pltpu.sync_copy(shared_avail, avail_v)
        plsc.subcore_barrier()

    @pl.when(c == 0)
    def _():
        for j in range(NROUNDS):
            pltpu.sync_copy(df_v.at[pl.ds(j * NCHUNK, NCHUNK)],
                            df_hbm.at[pl.ds(j * B + s * NCHUNK, NCHUNK)])


def _combine_body(rs_ref, df_ref, lg_ref, final_ref):
    rs = rs_ref[...]
    Df = df_ref[...]
    lg = lg_ref[...]

    active = rs * Df
    active = active - jnp.max(active, axis=1, keepdims=True)
    z = active + (Df - 1.0) * 1e9
    z = z - jnp.max(z, axis=1, keepdims=True)
    ez = jnp.exp(z)
    w = ez / jnp.sum(ez, axis=1, keepdims=True)

    rep_r = lax.broadcasted_iota(jnp.int32, (NUM_EXPERTS, NUM_EXPERTS * NCLS), 0)
    rep_c = lax.broadcasted_iota(jnp.int32, (NUM_EXPERTS, NUM_EXPERTS * NCLS), 1)
    R = jnp.where(rep_r == rep_c // NCLS, 1.0, 0.0)
    w_big = jnp.dot(w, R, preferred_element_type=jnp.float32)
    t = w_big * lg
    sum_r = lax.broadcasted_iota(jnp.int32, (NUM_EXPERTS * NCLS, NCLS), 0)
    sum_c = lax.broadcasted_iota(jnp.int32, (NUM_EXPERTS * NCLS, NCLS), 1)
    S = jnp.where(sum_r % NCLS == sum_c, 1.0, 0.0)
    final_ref[...] = jnp.dot(t, S, preferred_element_type=jnp.float32)


def _combine(rs, Df, lg, interpret=False):
    return pl.pallas_call(
        _combine_body,
        out_shape=jax.ShapeDtypeStruct((B, NCLS), jnp.float32),
        interpret=interpret,
    )(rs, Df, lg)


def kernel(x, params):
    feats = _trunk(x, params)
    lgT, rsT = _heads(feats.T, params)
    rs = rsT.T
    dft = _sc_route(rsT).reshape(NROUNDS, B)
    Df = jnp.concatenate(
        [dft.T, jnp.zeros((B, NUM_EXPERTS - NROUNDS), jnp.float32)], axis=1)
    final = _combine(rs, Df, lgT.T)
    return final, rs, Df.astype(bool)
